# baseline (speedup 1.0000x reference)
import functools
import jax
import jax.numpy as jnp
from jax import lax
from jax.experimental import pallas as pl
from jax.experimental.pallas import tpu as pltpu
from jax.experimental.pallas import tpu_sc as plsc

ROWS = 128
COLS = 32768
LANES = 16
NCHUNK = COLS // LANES
NWORKERS = 32
ROWS_PER_W = ROWS // NWORKERS


def _body(x_hbm, out_hbm, row_v, oh_v):
    c = lax.axis_index("c")
    s = lax.axis_index("s")
    wid = s * 2 + c

    lane_iota = lax.iota(jnp.int32, LANES)
    zeros16 = jnp.zeros((LANES,), jnp.float32)

    def zbody(i, carry):
        oh_v[pl.ds(i * LANES, LANES)] = zeros16
        return carry

    lax.fori_loop(0, NCHUNK, zbody, 0)

    for r in range(ROWS_PER_W):
        row = wid * ROWS_PER_W + r
        pltpu.sync_copy(x_hbm.at[row], row_v)

        def amax_body(i, carry):
            best, bidx = carry
            chunk = row_v[pl.ds(i * LANES, LANES)]
            m = chunk > best
            best = jnp.where(m, chunk, best)
            bidx = jnp.where(m, i * LANES + lane_iota, bidx)
            return best, bidx

        best0 = jnp.full((LANES,), -jnp.inf, jnp.float32)
        bidx0 = jnp.zeros((LANES,), jnp.int32)
        best, bidx = lax.fori_loop(0, NCHUNK, amax_body, (best0, bidx0))

        for shift in (8, 4, 2, 1):
            perm = (lane_iota + shift) & (LANES - 1)
            ov = best[perm]
            oi = bidx[perm]
            take = (ov > best) | ((ov == best) & (oi < bidx))
            best = jnp.where(take, ov, best)
            bidx = jnp.where(take, oi, bidx)
        idx = bidx[0]
        base = idx & ~(LANES - 1)
        hot = jnp.where(lane_iota == (idx - base), 1.0, 0.0).astype(jnp.float32)
        oh_v[pl.ds(base, LANES)] = hot
        pltpu.sync_copy(oh_v, out_hbm.at[row])
        oh_v[pl.ds(base, LANES)] = zeros16


@functools.partial(
    pl.kernel,
    out_type=jax.ShapeDtypeStruct((ROWS, COLS), jnp.float32),
    mesh=plsc.VectorSubcoreMesh(core_axis_name="c", subcore_axis_name="s"),
    scratch_types=[
        pltpu.VMEM((COLS,), jnp.float32),
        pltpu.VMEM((COLS,), jnp.float32),
    ],
)
def _onehot_argmax(x_hbm, out_hbm, row_v, oh_v):
    _body(x_hbm, out_hbm, row_v, oh_v)


def kernel(x):
    return _onehot_argmax(x)


# speedup vs baseline: 1.9852x; 1.9852x over previous
import functools
import jax
import jax.numpy as jnp
from jax import lax
from jax.experimental import pallas as pl
from jax.experimental.pallas import tpu as pltpu
from jax.experimental.pallas import tpu_sc as plsc

ROWS = 128
COLS = 32768
LANES = 16
NCHUNK = COLS // LANES
UNROLL = 8
NITER = NCHUNK // UNROLL
NWORKERS = 32
ROWS_PER_W = ROWS // NWORKERS

_NEG_INF = float("-inf")


def _combine(v_a, i_a, v_b, i_b):
    take_b = (v_b > v_a) | ((v_b == v_a) & (i_b < i_a))
    return jnp.where(take_b, v_b, v_a), jnp.where(take_b, i_b, i_a)


def _row_argmax(row_v, lane_iota):
    def body(i, carry):
        bests = list(carry[:UNROLL])
        bcnts = list(carry[UNROLL:])
        cbase = i * UNROLL
        for j in range(UNROLL):
            cnum = cbase + j
            chunk = row_v[pl.ds(cnum * LANES, LANES)]
            m = chunk > bests[j]
            bests[j] = jnp.where(m, chunk, bests[j])
            bcnts[j] = jnp.where(m, jnp.full((LANES,), cnum, jnp.int32), bcnts[j])
        return tuple(bests) + tuple(bcnts)

    init = tuple(jnp.full((LANES,), _NEG_INF, jnp.float32) for _ in range(UNROLL)) + tuple(
        jnp.zeros((LANES,), jnp.int32) for _ in range(UNROLL)
    )
    carry = lax.fori_loop(0, NITER, body, init)
    bests = list(carry[:UNROLL])
    gidxs = [carry[UNROLL + j] * LANES + lane_iota for j in range(UNROLL)]

    stride = 1
    while stride < UNROLL:
        for j in range(0, UNROLL, 2 * stride):
            bests[j], gidxs[j] = _combine(
                bests[j], gidxs[j], bests[j + stride], gidxs[j + stride]
            )
        stride *= 2
    best, gidx = bests[0], gidxs[0]

    for shift in (8, 4, 2, 1):
        perm = (lane_iota + shift) & (LANES - 1)
        best, gidx = _combine(best, gidx, best[perm], gidx[perm])
    return gidx


def _body(x_hbm, out_hbm, in0_v, in1_v, oh_v, semi0, semi1, semo):
    c = lax.axis_index("c")
    s = lax.axis_index("s")
    wid = s * 2 + c
    row0 = wid * ROWS_PER_W

    lane_iota = lax.iota(jnp.int32, LANES)
    zeros16 = jnp.zeros((LANES,), jnp.float32)

    in_bufs = (in0_v, in1_v)
    in_sems = (semi0, semi1)

    in_dma = [None] * ROWS_PER_W
    in_dma[0] = pltpu.async_copy(x_hbm.at[row0], in0_v, semi0)
    in_dma[1] = pltpu.async_copy(x_hbm.at[row0 + 1], in1_v, semi1)

    def zbody(i, carry):
        oh_v[pl.ds(i * LANES, LANES)] = zeros16
        return carry

    lax.fori_loop(0, NCHUNK, zbody, 0)

    out_dma = None
    for r in range(ROWS_PER_W):
        buf = in_bufs[r % 2]
        in_dma[r].wait()
        gidx = _row_argmax(buf, lane_iota)
        if r + 2 < ROWS_PER_W:
            in_dma[r + 2] = pltpu.async_copy(
                x_hbm.at[row0 + r + 2], buf, in_sems[r % 2]
            )
        idx = gidx[0]
        base = idx & ~(LANES - 1)
        hot = jnp.where(lane_iota == (idx - base), 1.0, 0.0).astype(jnp.float32)
        if out_dma is not None:
            out_dma.wait()
            oh_v[pl.ds(prev_base, LANES)] = zeros16
        oh_v[pl.ds(base, LANES)] = hot
        out_dma = pltpu.async_copy(oh_v, out_hbm.at[row0 + r], semo)
        prev_base = base
    out_dma.wait()


@functools.partial(
    pl.kernel,
    out_type=jax.ShapeDtypeStruct((ROWS, COLS), jnp.float32),
    mesh=plsc.VectorSubcoreMesh(core_axis_name="c", subcore_axis_name="s"),
    scratch_types=[
        pltpu.VMEM((COLS,), jnp.float32),
        pltpu.VMEM((COLS,), jnp.float32),
        pltpu.VMEM((COLS,), jnp.float32),
        pltpu.SemaphoreType.DMA,
        pltpu.SemaphoreType.DMA,
        pltpu.SemaphoreType.DMA,
    ],
)
def _onehot_argmax(x_hbm, out_hbm, in0_v, in1_v, oh_v, semi0, semi1, semo):
    _body(x_hbm, out_hbm, in0_v, in1_v, oh_v, semi0, semi1, semo)


def kernel(x):
    return _onehot_argmax(x)


# speedup vs baseline: 3.1723x; 1.5980x over previous
import functools
import jax
import jax.numpy as jnp
from jax.experimental import pallas as pl
from jax.experimental.pallas import tpu as pltpu

ROWS = 128
COLS = 32768
NBLK = 16
BLKC = COLS // NBLK
_BIG = 2**31 - 1


def _amax_body(x_ref, idx_ref, best_ref, bidx_ref):
    j = pl.program_id(0)

    @pl.when(j == 0)
    def _():
        best_ref[...] = jnp.full((ROWS, 1), float("-inf"), jnp.float32)
        bidx_ref[...] = jnp.zeros((ROWS, 1), jnp.int32)

    xb = x_ref[...]
    bmax = jnp.max(xb, axis=1, keepdims=True)
    cols = jax.lax.broadcasted_iota(jnp.int32, (ROWS, BLKC), 1)
    local = jnp.min(
        jnp.where(xb == bmax, cols, jnp.int32(_BIG)), axis=1, keepdims=True
    )
    better = bmax > best_ref[...]
    bidx_ref[...] = jnp.where(better, local + j * BLKC, bidx_ref[...])
    best_ref[...] = jnp.where(better, bmax, best_ref[...])

    @pl.when(j == NBLK - 1)
    def _():
        idx_ref[...] = bidx_ref[...]


_amax_call = pl.pallas_call(
    _amax_body,
    grid=(NBLK,),
    in_specs=[pl.BlockSpec((ROWS, BLKC), lambda j: (0, j))],
    out_specs=pl.BlockSpec((ROWS, 1), lambda j: (0, 0)),
    out_shape=jax.ShapeDtypeStruct((ROWS, 1), jnp.int32),
    scratch_shapes=[
        pltpu.VMEM((ROWS, 1), jnp.float32),
        pltpu.VMEM((ROWS, 1), jnp.int32),
    ],
)


def _oh_body(idx_ref, out_ref):
    j = pl.program_id(0)
    cols = jax.lax.broadcasted_iota(jnp.int32, (ROWS, BLKC), 1) + j * BLKC
    out_ref[...] = (cols == idx_ref[...]).astype(jnp.float32)


def kernel(x):
    idx = _amax_call(x)
    oh = pl.pallas_call(
        _oh_body,
        grid=(NBLK,),
        in_specs=[pl.BlockSpec((ROWS, 1), lambda j: (0, 0))],
        out_specs=pl.BlockSpec((ROWS, BLKC), lambda j: (0, j)),
        out_shape=jax.ShapeDtypeStruct((ROWS, COLS), jnp.float32),
    )(idx)
    return oh


# speedup vs baseline: 3.6923x; 1.1639x over previous
import jax
import jax.numpy as jnp
from jax.experimental import pallas as pl
from jax.experimental.pallas import tpu as pltpu

ROWS = 128
COLS = 32768
LANE = 128
NBLK = 16
BLKC = COLS // NBLK
CPB = BLKC // LANE
NBLKB = 8
BLKB = COLS // NBLKB
_BIG = 2**31 - 1


def _amax_body(x_ref, idx_ref, acc_ref, aidx_ref):
    j = pl.program_id(0)

    @pl.when(j == 0)
    def _():
        acc_ref[...] = jnp.full((ROWS, LANE), float("-inf"), jnp.float32)
        aidx_ref[...] = jnp.zeros((ROWS, LANE), jnp.int32)

    acc = acc_ref[...]
    aidx = aidx_ref[...]
    for c in range(CPB):
        xv = x_ref[:, c * LANE : (c + 1) * LANE]
        m = xv > acc
        acc = jnp.where(m, xv, acc)
        aidx = jnp.where(m, jnp.full((ROWS, LANE), j * CPB + c, jnp.int32), aidx)
    acc_ref[...] = acc
    aidx_ref[...] = aidx

    @pl.when(j == NBLK - 1)
    def _():
        rowmax = jnp.max(acc, axis=1, keepdims=True)
        lanes = jax.lax.broadcasted_iota(jnp.int32, (ROWS, LANE), 1)
        gidx = aidx * LANE + lanes
        idx_ref[...] = jnp.min(
            jnp.where(acc == rowmax, gidx, jnp.int32(_BIG)), axis=1, keepdims=True
        )


_amax_call = pl.pallas_call(
    _amax_body,
    grid=(NBLK,),
    in_specs=[pl.BlockSpec((ROWS, BLKC), lambda j: (0, j))],
    out_specs=pl.BlockSpec((ROWS, 1), lambda j: (0, 0)),
    out_shape=jax.ShapeDtypeStruct((ROWS, 1), jnp.int32),
    scratch_shapes=[
        pltpu.VMEM((ROWS, LANE), jnp.float32),
        pltpu.VMEM((ROWS, LANE), jnp.int32),
    ],
)


def _oh_body(idx_ref, out_ref):
    j = pl.program_id(0)
    il = idx_ref[...] - j * BLKB
    cols = jax.lax.broadcasted_iota(jnp.int32, (ROWS, BLKB), 1)
    out_ref[...] = jnp.where(cols == il, 1.0, 0.0).astype(jnp.float32)


_oh_call = pl.pallas_call(
    _oh_body,
    grid=(NBLKB,),
    in_specs=[pl.BlockSpec((ROWS, 1), lambda j: (0, 0))],
    out_specs=pl.BlockSpec((ROWS, BLKB), lambda j: (0, j)),
    out_shape=jax.ShapeDtypeStruct((ROWS, COLS), jnp.float32),
)


def kernel(x):
    return _oh_call(_amax_call(x))


# speedup vs baseline: 5.2253x; 1.4152x over previous
import jax
import jax.numpy as jnp
from jax.experimental import pallas as pl

ROWS = 128
COLS = 32768
LANE = 128
RPB = 16
NB = ROWS // RPB
CV = COLS // LANE
_BIG = 2**31 - 1


def _body(x_ref, out_ref):
    acc = x_ref[:, 0:LANE]
    aidx = jnp.zeros((RPB, LANE), jnp.int32)
    for c in range(1, CV):
        xv = x_ref[:, c * LANE : (c + 1) * LANE]
        m = xv > acc
        acc = jnp.where(m, xv, acc)
        aidx = jnp.where(m, jnp.full((RPB, LANE), c, jnp.int32), aidx)
    rowmax = jnp.max(acc, axis=1, keepdims=True)
    lanes = jax.lax.broadcasted_iota(jnp.int32, (RPB, LANE), 1)
    gidx = aidx * LANE + lanes
    idx = jnp.min(
        jnp.where(acc == rowmax, gidx, jnp.int32(_BIG)), axis=1, keepdims=True
    )
    cols = jax.lax.broadcasted_iota(jnp.int32, (RPB, COLS), 1)
    out_ref[...] = jnp.where(cols == idx, 1.0, 0.0).astype(jnp.float32)


_call = pl.pallas_call(
    _body,
    grid=(NB,),
    in_specs=[pl.BlockSpec((RPB, COLS), lambda j: (j, 0))],
    out_specs=pl.BlockSpec((RPB, COLS), lambda j: (j, 0)),
    out_shape=jax.ShapeDtypeStruct((ROWS, COLS), jnp.float32),
)


def kernel(x):
    return _call(x)
